# baseline (speedup 1.0000x reference)
import functools

import jax
import jax.numpy as jnp
from jax import lax
from jax.experimental import pallas as pl
from jax.experimental.pallas import tpu as pltpu
from jax.experimental.pallas import tpu_sc as plsc

B = 16
P = 1516
C_IN = 3
F = 64
N = B * P
E = N * 16

NC = 2
NS = 16
NW = NC * NS
CH = 128
CPW = 96
BL = 12
NB = CPW // BL
EPW = CPW * CH
E_PAD = NW * EPW
RPS = 1520
NP = NS * RPS

_mesh = plsc.VectorSubcoreMesh(core_axis_name="c", subcore_axis_name="s")

_GATHER_DNUMS = lax.GatherDimensionNumbers(
    offset_dims=(), collapsed_slice_dims=(0,), start_index_map=(0,))


def _bcast_lane(vv, l):
    idx = jnp.full((16, 1), l, jnp.int32)
    return lax.gather(vv, idx, _GATHER_DNUMS, (1,),
                      mode=lax.GatherScatterMode.PROMISE_IN_BOUNDS)


FH = F // 2


@functools.partial(
    pl.kernel,
    out_type=jax.ShapeDtypeStruct((NC, 2, NP, FH), jnp.float32),
    mesh=_mesh,
    scratch_types=[
        pltpu.VMEM((BL, CH), jnp.int32),
        pltpu.VMEM((BL, CH), jnp.int32),
        pltpu.VMEM((BL * CH,), jnp.float32),
        pltpu.VMEM((CH, FH // 2), jnp.int32),
        pltpu.VMEM((CH, FH // 2), jnp.int32),
        pltpu.VMEM((CH, FH // 2), jnp.int32),
        pltpu.VMEM((CH, FH), jnp.float32),
        pltpu.VMEM((CH, FH), jnp.float32),
        pltpu.VMEM((CH, FH), jnp.float32),
        pltpu.VMEM_SHARED((NP, FH // 2), jnp.int32),
        pltpu.VMEM_SHARED((NP, FH), jnp.float32),
        pltpu.SemaphoreType.DMA,
        pltpu.SemaphoreType.DMA,
    ],
    compiler_params=pltpu.CompilerParams(use_tc_tiling_on_sc=False),
)
def _spmm_sc(h_hbm, src_hbm, dst_hbm, vals_hbm, zeros_hbm, out_hbm,
             src_v, dst_v, vals_v, rows0, rows1, rows2,
             msg0, msg1, msg2, hspm, acc, gsem, ssem):
    c = lax.axis_index("c")
    s = lax.axis_index("s")
    wid = s * NC + c

    def start_gather(jj, rows_ref):
        pltpu.async_copy(hspm.at[src_v.at[jj]], rows_ref, gsem)

    def wait_gather(jj, rows_ref):
        pltpu.make_async_copy(hspm.at[src_v.at[jj]], rows_ref, gsem).wait()

    def start_scatter(jj, msg_ref):
        pltpu.async_copy(msg_ref, acc.at[dst_v.at[jj]], ssem, add=True)

    def wait_scatter(jj, msg_ref):
        pltpu.make_async_copy(msg_ref, acc.at[dst_v.at[jj]],
                              ssem).wait()

    def scale(rows_ref, msg_ref, j):
        def scale_body(g, carry2):
            vv = vals_v[pl.ds(j * CH + g * 16, 16)]
            for l in range(16):
                v16 = _bcast_lane(vv, l)
                i = g * 16 + l
                w = rows_ref[i, :]
                lo = lax.bitcast_convert_type(
                    lax.shift_left(w, 16), jnp.float32)
                hi = lax.bitcast_convert_type(
                    lax.bitwise_and(w, jnp.int32(-65536)), jnp.float32)
                msg_ref[i, pl.ds(0, 16)] = lo * v16
                msg_ref[i, pl.ds(16, 16)] = hi * v16
            return carry2

        lax.fori_loop(0, CH // 16, scale_body, 0)

    def step(t, cur, cur_msg, prev, prev_msg):
        wait_gather(t, cur)
        scale(cur, cur_msg, t)
        start_scatter(t, cur_msg)

        @pl.when(t >= 1)
        def _():
            wait_scatter(t - 1, prev_msg)

        @pl.when(t + 2 < BL)
        def _():
            start_gather(t + 2, prev)

    def block_body(blk, carry):
        pltpu.sync_copy(src_hbm.at[wid, pl.ds(blk * BL, BL)], src_v)
        pltpu.sync_copy(dst_hbm.at[wid, pl.ds(blk * BL, BL)], dst_v)
        pltpu.sync_copy(vals_hbm.at[wid, pl.ds(blk * BL * CH, BL * CH)],
                        vals_v)
        start_gather(0, rows0)
        start_gather(1, rows1)

        def inner(m, carry1):
            t = 3 * m
            step(t, rows0, msg0, rows2, msg2)
            step(t + 1, rows1, msg1, rows0, msg0)
            step(t + 2, rows2, msg2, rows1, msg1)
            return carry1

        lax.fori_loop(0, BL // 3, inner, 0)
        wait_scatter(BL - 1, msg2)
        return carry

    for ph in range(2):
        pltpu.sync_copy(h_hbm.at[ph, pl.ds(s * RPS, RPS)],
                        hspm.at[pl.ds(s * RPS, RPS)])
        pltpu.sync_copy(zeros_hbm.at[pl.ds(s * RPS, RPS)],
                        acc.at[pl.ds(s * RPS, RPS)])
        plsc.subcore_barrier()

        lax.fori_loop(0, NB, block_body, 0)

        plsc.subcore_barrier()
        pltpu.sync_copy(acc.at[pl.ds(s * RPS, RPS)],
                        out_hbm.at[c, ph, pl.ds(s * RPS, RPS)])


NK = N * FH // 128
NPK = NP * FH // 128


def _bn_packed(a, gp, bep):
    m4 = jnp.mean(a, axis=0, keepdims=True)
    s4 = jnp.mean(a * a, axis=0, keepdims=True)
    mg = (m4[:, 0:32] + m4[:, 32:64] + m4[:, 64:96] + m4[:, 96:128]) * 0.25
    sg = (s4[:, 0:32] + s4[:, 32:64] + s4[:, 64:96] + s4[:, 96:128]) * 0.25
    var = sg - mg * mg
    scale = lax.rsqrt(var + 1e-5)
    mg = jnp.concatenate([mg, mg, mg, mg], axis=1)
    scale = jnp.concatenate([scale, scale, scale, scale], axis=1)
    return jnp.maximum((a - mg) * scale * gp + bep, 0.0)


def _mm_body(x_ref, wlo_ref, whi_ref, o_ref):
    xb = x_ref[...]
    o_ref[0, :NK, :] = jnp.dot(xb, wlo_ref[...],
                               preferred_element_type=jnp.float32)
    o_ref[1, :NK, :] = jnp.dot(xb, whi_ref[...],
                               preferred_element_type=jnp.float32)


def _mid_body(q_ref, wll_ref, whl_ref, wlh_ref, whh_ref,
              glo_ref, ghi_ref, blo_ref, bhi_ref, o_ref):
    a = q_ref[0, :NK, :] + q_ref[2, :NK, :]
    b = q_ref[1, :NK, :] + q_ref[3, :NK, :]
    an = _bn_packed(a, glo_ref[...], blo_ref[...])
    bn_ = _bn_packed(b, ghi_ref[...], bhi_ref[...])
    o_ref[0, :NK, :] = (
        jnp.dot(an, wll_ref[...], preferred_element_type=jnp.float32)
        + jnp.dot(bn_, whl_ref[...], preferred_element_type=jnp.float32))
    o_ref[1, :NK, :] = (
        jnp.dot(an, wlh_ref[...], preferred_element_type=jnp.float32)
        + jnp.dot(bn_, whh_ref[...], preferred_element_type=jnp.float32))


def _last_body(q_ref, glo_ref, ghi_ref, blo_ref, bhi_ref, o_ref):
    a = q_ref[0, :NK, :] + q_ref[2, :NK, :]
    b = q_ref[1, :NK, :] + q_ref[3, :NK, :]
    o_ref[0, :, :] = _bn_packed(a, glo_ref[...], blo_ref[...])
    o_ref[1, :, :] = _bn_packed(b, ghi_ref[...], bhi_ref[...])


KB = 6064


def _cls_body(a_ref, w_ref, gc_ref, bec_ref, wc2_ref, bc2_ref, o_ref, acc_ref):
    k = pl.program_id(0)

    @pl.when(k == 0)
    def _():
        acc_ref[...] = jnp.zeros_like(acc_ref)

    acc_ref[...] += lax.dot_general(
        a_ref[...], w_ref[...], (((0,), (0,)), ((), ())),
        preferred_element_type=jnp.float32)

    @pl.when(k == pl.num_programs(0) - 1)
    def _():
        z = acc_ref[...]
        m = jnp.mean(z, axis=0, keepdims=True)
        d = z - m
        v = jnp.mean(d * d, axis=0, keepdims=True)
        zn = d * lax.rsqrt(v + 1e-5) * gc_ref[...] + bec_ref[...]
        zn = jnp.maximum(zn, 0.0)
        o_ref[...] = (jnp.dot(zn, wc2_ref[...],
                              preferred_element_type=jnp.float32)
                      + bc2_ref[...])


def _bd4(a):
    return jnp.kron(jnp.eye(4, dtype=a.dtype), a)


def _tile4(v):
    return jnp.tile(v, 4).reshape(1, 128)


def _mm(x, w):
    xp = x.reshape(NK, 32)
    return pl.pallas_call(
        _mm_body,
        out_shape=jax.ShapeDtypeStruct((2, NPK, 128), jnp.float32),
    )(xp, _bd4(w[:, :FH]), _bd4(w[:, FH:]))


def _mid(p, g, be, w):
    q = p.reshape(4, NPK, 128)
    return pl.pallas_call(
        _mid_body,
        out_shape=jax.ShapeDtypeStruct((2, NPK, 128), jnp.float32),
    )(q, _bd4(w[:FH, :FH]), _bd4(w[FH:, :FH]),
      _bd4(w[:FH, FH:]), _bd4(w[FH:, FH:]),
      _tile4(g[:FH]), _tile4(g[FH:]), _tile4(be[:FH]), _tile4(be[FH:]))


def _last(p, g, be):
    q = p.reshape(4, NPK, 128)
    return pl.pallas_call(
        _last_body,
        out_shape=jax.ShapeDtypeStruct((2, NK, 128), jnp.float32),
    )(q, _tile4(g[:FH]), _tile4(g[FH:]), _tile4(be[:FH]), _tile4(be[FH:]))


def _classifier(at, wc1, gc, bec, wc2, bc2):
    nk = at.shape[0] // KB
    return pl.pallas_call(
        _cls_body,
        grid=(nk,),
        in_specs=[
            pl.BlockSpec((KB, B), lambda k: (k, 0)),
            pl.BlockSpec((KB, 128), lambda k: (k, 0)),
            pl.BlockSpec((1, 128), lambda k: (0, 0)),
            pl.BlockSpec((1, 128), lambda k: (0, 0)),
            pl.BlockSpec((128, 2), lambda k: (0, 0)),
            pl.BlockSpec((1, 2), lambda k: (0, 0)),
        ],
        out_specs=pl.BlockSpec((B, 2), lambda k: (0, 0)),
        out_shape=jax.ShapeDtypeStruct((B, 2), jnp.float32),
        scratch_shapes=[pltpu.VMEM((B, 128), jnp.float32)],
    )(at, wc1, gc.reshape(1, 128), bec.reshape(1, 128), wc2,
      bc2.reshape(1, 2))


def kernel(x, edge_index, edge_vals, W1, b1, g1, be1, W2, b2, g2, be2,
           W3, b3, g3, be3, Wc1, bc1, gc, bec, Wc2, bc2):
    ht = jnp.transpose(x, (0, 2, 1)).reshape(N, C_IN)
    ht8 = jnp.pad(ht, ((0, 0), (0, 8 - C_IN)))
    W1p = jnp.pad(W1, ((0, 8 - C_IN), (0, 0)))

    pad = E_PAD - E
    src = jnp.concatenate([edge_index[0], jnp.zeros((pad,), jnp.int32)])
    dst = jnp.concatenate([edge_index[1], jnp.zeros((pad,), jnp.int32)])
    vals = jnp.concatenate([edge_vals, jnp.zeros((pad,), jnp.float32)])
    src = src.reshape(NW, CPW, CH)
    dst = dst.reshape(NW, CPW, CH)
    vals = vals.reshape(NW, EPW)
    zeros = jnp.zeros((NP, FH), jnp.float32)

    mix = jnp.array([x for i in range(16) for x in (i, 16 + i)], jnp.int32)

    def to_bf(h):
        hb = h[:, :, mix].astype(jnp.bfloat16)
        return lax.bitcast_convert_type(
            hb.reshape(2, NP, FH // 2, 2), jnp.int32)

    h = _mm(ht8, W1p).reshape(2, NP, FH)
    p = _spmm_sc(to_bf(h), src, dst, vals, zeros)
    h = _mid(p, g1, be1, W2).reshape(2, NP, FH)
    p = _spmm_sc(to_bf(h), src, dst, vals, zeros)
    h = _mid(p, g2, be2, W3).reshape(2, NP, FH)
    p = _spmm_sc(to_bf(h), src, dst, vals, zeros)
    hn = _last(p, g3, be3)

    h3 = jnp.concatenate(
        [hn[0].reshape(N, FH), hn[1].reshape(N, FH)], axis=1)
    at = h3.reshape(B, P * F).T
    return _classifier(at, Wc1, gc, bec, Wc2, bc2)


# speedup vs baseline: 1.2229x; 1.2229x over previous
import functools

import jax
import jax.numpy as jnp
from jax import lax
from jax.experimental import pallas as pl
from jax.experimental.pallas import tpu as pltpu
from jax.experimental.pallas import tpu_sc as plsc

B = 16
P = 1516
C_IN = 3
F = 64
N = B * P
E = N * 16

NC = 2
NS = 16
NW = NC * NS
CH = 128
CPW = 96
BL = 12
NB = CPW // BL
EPW = CPW * CH
E_PAD = NW * EPW
RPS = 1520
NP = NS * RPS

_mesh = plsc.VectorSubcoreMesh(core_axis_name="c", subcore_axis_name="s")

_GATHER_DNUMS = lax.GatherDimensionNumbers(
    offset_dims=(), collapsed_slice_dims=(0,), start_index_map=(0,))


def _bcast_lane(vv, l):
    idx = jnp.full((16, 1), l, jnp.int32)
    return lax.gather(vv, idx, _GATHER_DNUMS, (1,),
                      mode=lax.GatherScatterMode.PROMISE_IN_BOUNDS)


FH = F // 2


@functools.partial(
    pl.kernel,
    out_type=jax.ShapeDtypeStruct((NC, 2, NP, FH), jnp.float32),
    mesh=_mesh,
    scratch_types=[
        pltpu.VMEM((BL, CH), jnp.int32),
        pltpu.VMEM((BL, CH), jnp.int32),
        pltpu.VMEM((BL * CH,), jnp.float32),
        pltpu.VMEM((CH, FH), jnp.float32),
        pltpu.VMEM((CH, FH), jnp.float32),
        pltpu.VMEM((CH, FH), jnp.float32),
        pltpu.VMEM_SHARED((NP, FH), jnp.float32),
        pltpu.VMEM_SHARED((NP, FH), jnp.float32),
        pltpu.SemaphoreType.DMA,
        pltpu.SemaphoreType.DMA,
        pltpu.SemaphoreType.DMA,
    ],
    compiler_params=pltpu.CompilerParams(use_tc_tiling_on_sc=False),
)
def _spmm_sc(h0_hbm, h1_hbm, src_hbm, dst_hbm, vals_hbm, zeros_hbm, out_hbm,
             src_v, dst_v, vals_v, rows0, rows1, rows2, hspm, acc,
             gsem, ssem, hsem):
    c = lax.axis_index("c")
    s = lax.axis_index("s")
    wid = s * NC + c


    def start_scatter(jj, rows_ref):
        pltpu.async_copy(rows_ref, acc.at[dst_v.at[jj]], ssem, add=True)

    def wait_scatter(jj, rows_ref):
        pltpu.make_async_copy(rows_ref, acc.at[dst_v.at[jj]],
                              ssem).wait()

    def scale(rows_ref, j):
        def scale_body(g, carry2):
            vv = vals_v[pl.ds(j * CH + g * 16, 16)]
            for l in range(16):
                v16 = _bcast_lane(vv, l)
                i = g * 16 + l
                for cb in range(FH // 16):
                    sl = (i, pl.ds(cb * 16, 16))
                    rows_ref[sl] = rows_ref[sl] * v16
            return carry2

        lax.fori_loop(0, CH // 16, scale_body, 0)

    def step(t, cur, prev, wait_cur, start_prev):
        wait_cur(t, cur)
        scale(cur, t)
        start_scatter(t, cur)

        @pl.when(t >= 1)
        def _():
            wait_scatter(t - 1, prev)

        @pl.when(t + 2 < BL)
        def _():
            start_prev(t + 2, prev)

    def make_block_body(ph):
        hph = h0_hbm if ph == 0 else h1_hbm
        def start_spm(jj, rows_ref):
            pltpu.async_copy(hspm.at[src_v.at[jj]], rows_ref, gsem)

        def wait_spm(jj, rows_ref):
            pltpu.make_async_copy(hspm.at[src_v.at[jj]], rows_ref,
                                  gsem).wait()

        def start_hbm(jj, rows_ref):
            pltpu.async_copy(hph.at[src_v.at[jj]], rows_ref, hsem)

        def wait_hbm(jj, rows_ref):
            pltpu.make_async_copy(hph.at[src_v.at[jj]], rows_ref,
                                  hsem).wait()

        def block_body(blk, carry):
            pltpu.sync_copy(src_hbm.at[wid, pl.ds(blk * BL, BL)], src_v)
            pltpu.sync_copy(dst_hbm.at[wid, pl.ds(blk * BL, BL)], dst_v)
            pltpu.sync_copy(vals_hbm.at[wid, pl.ds(blk * BL * CH, BL * CH)],
                            vals_v)
            start_spm(0, rows0)
            start_spm(1, rows1)

            def inner(m, carry1):
                t = 3 * m
                step(t, rows0, rows2, wait_spm, start_hbm)
                step(t + 1, rows1, rows0, wait_spm, start_spm)
                step(t + 2, rows2, rows1, wait_hbm, start_spm)
                return carry1

            lax.fori_loop(0, BL // 3, inner, 0)
            wait_scatter(BL - 1, rows2)
            return carry

        return block_body

    for ph in range(2):
        pltpu.sync_copy((h0_hbm if ph == 0 else h1_hbm).at[pl.ds(s * RPS,
                                                                  RPS)],
                        hspm.at[pl.ds(s * RPS, RPS)])
        pltpu.sync_copy(zeros_hbm.at[pl.ds(s * RPS, RPS)],
                        acc.at[pl.ds(s * RPS, RPS)])
        plsc.subcore_barrier()

        lax.fori_loop(0, NB, make_block_body(ph), 0)

        plsc.subcore_barrier()
        pltpu.sync_copy(acc.at[pl.ds(s * RPS, RPS)],
                        out_hbm.at[c, ph, pl.ds(s * RPS, RPS)])


NK = N * FH // 128
NPK = NP * FH // 128


def _bn_packed(a, gp, bep):
    m4 = jnp.mean(a, axis=0, keepdims=True)
    s4 = jnp.mean(a * a, axis=0, keepdims=True)
    mg = (m4[:, 0:32] + m4[:, 32:64] + m4[:, 64:96] + m4[:, 96:128]) * 0.25
    sg = (s4[:, 0:32] + s4[:, 32:64] + s4[:, 64:96] + s4[:, 96:128]) * 0.25
    var = sg - mg * mg
    scale = lax.rsqrt(var + 1e-5)
    mg = jnp.concatenate([mg, mg, mg, mg], axis=1)
    scale = jnp.concatenate([scale, scale, scale, scale], axis=1)
    return jnp.maximum((a - mg) * scale * gp + bep, 0.0)


def _mm_body(x_ref, wlo_ref, whi_ref, o_ref):
    xb = x_ref[...]
    o_ref[0, :NK, :] = jnp.dot(xb, wlo_ref[...],
                               preferred_element_type=jnp.float32)
    o_ref[1, :NK, :] = jnp.dot(xb, whi_ref[...],
                               preferred_element_type=jnp.float32)


def _mid_body(q_ref, wll_ref, whl_ref, wlh_ref, whh_ref,
              glo_ref, ghi_ref, blo_ref, bhi_ref, o_ref):
    a = q_ref[0, :NK, :] + q_ref[2, :NK, :]
    b = q_ref[1, :NK, :] + q_ref[3, :NK, :]
    an = _bn_packed(a, glo_ref[...], blo_ref[...])
    bn_ = _bn_packed(b, ghi_ref[...], bhi_ref[...])
    o_ref[0, :NK, :] = (
        jnp.dot(an, wll_ref[...], preferred_element_type=jnp.float32)
        + jnp.dot(bn_, whl_ref[...], preferred_element_type=jnp.float32))
    o_ref[1, :NK, :] = (
        jnp.dot(an, wlh_ref[...], preferred_element_type=jnp.float32)
        + jnp.dot(bn_, whh_ref[...], preferred_element_type=jnp.float32))


def _last_body(q_ref, glo_ref, ghi_ref, blo_ref, bhi_ref, o_ref):
    a = q_ref[0, :NK, :] + q_ref[2, :NK, :]
    b = q_ref[1, :NK, :] + q_ref[3, :NK, :]
    o_ref[0, :, :] = _bn_packed(a, glo_ref[...], blo_ref[...])
    o_ref[1, :, :] = _bn_packed(b, ghi_ref[...], bhi_ref[...])


KB = 6064


def _cls_body(a_ref, w_ref, gc_ref, bec_ref, wc2_ref, bc2_ref, o_ref, acc_ref):
    k = pl.program_id(0)

    @pl.when(k == 0)
    def _():
        acc_ref[...] = jnp.zeros_like(acc_ref)

    acc_ref[...] += lax.dot_general(
        a_ref[...], w_ref[...], (((0,), (0,)), ((), ())),
        preferred_element_type=jnp.float32)

    @pl.when(k == pl.num_programs(0) - 1)
    def _():
        z = acc_ref[...]
        m = jnp.mean(z, axis=0, keepdims=True)
        d = z - m
        v = jnp.mean(d * d, axis=0, keepdims=True)
        zn = d * lax.rsqrt(v + 1e-5) * gc_ref[...] + bec_ref[...]
        zn = jnp.maximum(zn, 0.0)
        o_ref[...] = (jnp.dot(zn, wc2_ref[...],
                              preferred_element_type=jnp.float32)
                      + bc2_ref[...])


def _bd4(a):
    return jnp.kron(jnp.eye(4, dtype=a.dtype), a)


def _tile4(v):
    return jnp.tile(v, 4).reshape(1, 128)


def _mm(x, w):
    xp = x.reshape(NK, 32)
    return pl.pallas_call(
        _mm_body,
        out_shape=jax.ShapeDtypeStruct((2, NPK, 128), jnp.float32),
    )(xp, _bd4(w[:, :FH]), _bd4(w[:, FH:]))


def _mid(p, g, be, w):
    q = p.reshape(4, NPK, 128)
    return pl.pallas_call(
        _mid_body,
        out_shape=jax.ShapeDtypeStruct((2, NPK, 128), jnp.float32),
    )(q, _bd4(w[:FH, :FH]), _bd4(w[FH:, :FH]),
      _bd4(w[:FH, FH:]), _bd4(w[FH:, FH:]),
      _tile4(g[:FH]), _tile4(g[FH:]), _tile4(be[:FH]), _tile4(be[FH:]))


def _last(p, g, be):
    q = p.reshape(4, NPK, 128)
    return pl.pallas_call(
        _last_body,
        out_shape=jax.ShapeDtypeStruct((2, NK, 128), jnp.float32),
    )(q, _tile4(g[:FH]), _tile4(g[FH:]), _tile4(be[:FH]), _tile4(be[FH:]))


def _classifier(at, wc1, gc, bec, wc2, bc2):
    nk = at.shape[0] // KB
    return pl.pallas_call(
        _cls_body,
        grid=(nk,),
        in_specs=[
            pl.BlockSpec((KB, B), lambda k: (k, 0)),
            pl.BlockSpec((KB, 128), lambda k: (k, 0)),
            pl.BlockSpec((1, 128), lambda k: (0, 0)),
            pl.BlockSpec((1, 128), lambda k: (0, 0)),
            pl.BlockSpec((128, 2), lambda k: (0, 0)),
            pl.BlockSpec((1, 2), lambda k: (0, 0)),
        ],
        out_specs=pl.BlockSpec((B, 2), lambda k: (0, 0)),
        out_shape=jax.ShapeDtypeStruct((B, 2), jnp.float32),
        scratch_shapes=[pltpu.VMEM((B, 128), jnp.float32)],
    )(at, wc1, gc.reshape(1, 128), bec.reshape(1, 128), wc2,
      bc2.reshape(1, 2))


def kernel(x, edge_index, edge_vals, W1, b1, g1, be1, W2, b2, g2, be2,
           W3, b3, g3, be3, Wc1, bc1, gc, bec, Wc2, bc2):
    ht = jnp.transpose(x, (0, 2, 1)).reshape(N, C_IN)
    ht8 = jnp.pad(ht, ((0, 0), (0, 8 - C_IN)))
    W1p = jnp.pad(W1, ((0, 8 - C_IN), (0, 0)))

    pad = E_PAD - E
    src = jnp.concatenate([edge_index[0], jnp.zeros((pad,), jnp.int32)])
    dst = jnp.concatenate([edge_index[1], jnp.zeros((pad,), jnp.int32)])
    vals = jnp.concatenate([edge_vals, jnp.zeros((pad,), jnp.float32)])
    src = src.reshape(NW, CPW, CH)
    dst = dst.reshape(NW, CPW, CH)
    vals = vals.reshape(NW, EPW)
    zeros = jnp.zeros((NP, FH), jnp.float32)

    h = _mm(ht8, W1p).reshape(2, NP, FH)
    p = _spmm_sc(h[0], h[1], src, dst, vals, zeros)
    h = _mid(p, g1, be1, W2).reshape(2, NP, FH)
    p = _spmm_sc(h[0], h[1], src, dst, vals, zeros)
    h = _mid(p, g2, be2, W3).reshape(2, NP, FH)
    p = _spmm_sc(h[0], h[1], src, dst, vals, zeros)
    hn = _last(p, g3, be3)

    h3 = jnp.concatenate(
        [hn[0].reshape(N, FH), hn[1].reshape(N, FH)], axis=1)
    at = h3.reshape(B, P * F).T
    return _classifier(at, Wc1, gc, bec, Wc2, bc2)


# speedup vs baseline: 1.8483x; 1.5114x over previous
import functools

import jax
import jax.numpy as jnp
from jax import lax
from jax.experimental import pallas as pl
from jax.experimental.pallas import tpu as pltpu
from jax.experimental.pallas import tpu_sc as plsc

B = 16
P = 1516
C_IN = 3
F = 64
N = B * P
E = N * 16

NC = 2
NS = 16
NW = NC * NS
CH = 128
CPW = 96
BL = 24
NB = CPW // BL
EPW = CPW * CH
E_PAD = NW * EPW
RPS = 1520
NP = NS * RPS

_mesh = plsc.VectorSubcoreMesh(core_axis_name="c", subcore_axis_name="s")

_GATHER_DNUMS = lax.GatherDimensionNumbers(
    offset_dims=(), collapsed_slice_dims=(0,), start_index_map=(0,))


def _bcast_lane(vv, l):
    idx = jnp.full((16, 1), l, jnp.int32)
    return lax.gather(vv, idx, _GATHER_DNUMS, (1,),
                      mode=lax.GatherScatterMode.PROMISE_IN_BOUNDS)


FH = F // 2


@functools.partial(
    pl.kernel,
    out_type=jax.ShapeDtypeStruct((NC, 2, NP, FH), jnp.float32),
    mesh=_mesh,
    scratch_types=[
        pltpu.VMEM((BL, CH), jnp.int32),
        pltpu.VMEM((BL, CH), jnp.int32),
        pltpu.VMEM((BL * CH,), jnp.float32),
        pltpu.VMEM((CH, FH), jnp.float32),
        pltpu.VMEM((CH, FH), jnp.float32),
        pltpu.VMEM((CH, FH), jnp.float32),
        pltpu.VMEM_SHARED((NP, FH), jnp.float32),
        pltpu.VMEM_SHARED((NP, FH), jnp.float32),
        pltpu.SemaphoreType.DMA,
        pltpu.SemaphoreType.DMA,
    ],
    compiler_params=pltpu.CompilerParams(use_tc_tiling_on_sc=False),
)
def _spmm_sc(h_hbm, src_hbm, dst_hbm, vals_hbm, zeros_hbm, out_hbm,
             src_v, dst_v, vals_v, rows0, rows1, rows2, hspm, acc,
             gsem, ssem):
    c = lax.axis_index("c")
    s = lax.axis_index("s")
    wid = s * NC + c

    def start_gather(jj, rows_ref):
        pltpu.async_copy(hspm.at[src_v.at[jj]], rows_ref, gsem)

    def wait_gather(jj, rows_ref):
        pltpu.make_async_copy(hspm.at[src_v.at[jj]], rows_ref, gsem).wait()

    def start_scatter(jj, rows_ref):
        pltpu.async_copy(rows_ref, acc.at[dst_v.at[jj]], ssem, add=True)

    def wait_scatter(jj, rows_ref):
        pltpu.make_async_copy(rows_ref, acc.at[dst_v.at[jj]],
                              ssem).wait()

    def scale(rows_ref, j):
        def scale_body(g, carry2):
            vv = vals_v[pl.ds(j * CH + g * 16, 16)]
            for l in range(16):
                v16 = _bcast_lane(vv, l)
                i = g * 16 + l
                for cb in range(FH // 16):
                    sl = (i, pl.ds(cb * 16, 16))
                    rows_ref[sl] = rows_ref[sl] * v16
            return carry2

        lax.fori_loop(0, CH // 16, scale_body, 0)

    def step(t, cur, prev):
        wait_gather(t, cur)
        scale(cur, t)
        start_scatter(t, cur)

        @pl.when(t >= 1)
        def _():
            wait_scatter(t - 1, prev)

        @pl.when(t + 2 < BL)
        def _():
            start_gather(t + 2, prev)

    def block_body(blk, carry):
        pltpu.sync_copy(src_hbm.at[wid, pl.ds(blk * BL, BL)], src_v)
        pltpu.sync_copy(dst_hbm.at[wid, pl.ds(blk * BL, BL)], dst_v)
        pltpu.sync_copy(vals_hbm.at[wid, pl.ds(blk * BL * CH, BL * CH)],
                        vals_v)
        start_gather(0, rows0)
        start_gather(1, rows1)

        def inner(m, carry1):
            t = 3 * m
            step(t, rows0, rows2)
            step(t + 1, rows1, rows0)
            step(t + 2, rows2, rows1)
            return carry1

        lax.fori_loop(0, BL // 3, inner, 0)
        wait_scatter(BL - 1, rows2)
        return carry

    for ph in range(2):
        pltpu.sync_copy(h_hbm.at[ph, pl.ds(s * RPS, RPS)],
                        hspm.at[pl.ds(s * RPS, RPS)])
        pltpu.sync_copy(zeros_hbm.at[pl.ds(s * RPS, RPS)],
                        acc.at[pl.ds(s * RPS, RPS)])
        plsc.subcore_barrier()

        lax.fori_loop(0, NB, block_body, 0)

        plsc.subcore_barrier()
        pltpu.sync_copy(acc.at[pl.ds(s * RPS, RPS)],
                        out_hbm.at[c, ph, pl.ds(s * RPS, RPS)])


NK = N * FH // 128
NPK = NP * FH // 128


def _bn_packed(a, gp, bep):
    m4 = jnp.mean(a, axis=0, keepdims=True)
    s4 = jnp.mean(a * a, axis=0, keepdims=True)
    mg = (m4[:, 0:32] + m4[:, 32:64] + m4[:, 64:96] + m4[:, 96:128]) * 0.25
    sg = (s4[:, 0:32] + s4[:, 32:64] + s4[:, 64:96] + s4[:, 96:128]) * 0.25
    var = sg - mg * mg
    scale = lax.rsqrt(var + 1e-5)
    mg = jnp.concatenate([mg, mg, mg, mg], axis=1)
    scale = jnp.concatenate([scale, scale, scale, scale], axis=1)
    return jnp.maximum((a - mg) * scale * gp + bep, 0.0)


def _mm_body(x_ref, wlo_ref, whi_ref, o_ref):
    xb = x_ref[...]
    o_ref[0, :NK, :] = jnp.dot(xb, wlo_ref[...],
                               preferred_element_type=jnp.float32)
    o_ref[1, :NK, :] = jnp.dot(xb, whi_ref[...],
                               preferred_element_type=jnp.float32)


def _mid_body(q_ref, wll_ref, whl_ref, wlh_ref, whh_ref,
              glo_ref, ghi_ref, blo_ref, bhi_ref, o_ref):
    a = q_ref[0, :NK, :] + q_ref[2, :NK, :]
    b = q_ref[1, :NK, :] + q_ref[3, :NK, :]
    an = _bn_packed(a, glo_ref[...], blo_ref[...])
    bn_ = _bn_packed(b, ghi_ref[...], bhi_ref[...])
    o_ref[0, :NK, :] = (
        jnp.dot(an, wll_ref[...], preferred_element_type=jnp.float32)
        + jnp.dot(bn_, whl_ref[...], preferred_element_type=jnp.float32))
    o_ref[1, :NK, :] = (
        jnp.dot(an, wlh_ref[...], preferred_element_type=jnp.float32)
        + jnp.dot(bn_, whh_ref[...], preferred_element_type=jnp.float32))


def _last_body(q_ref, glo_ref, ghi_ref, blo_ref, bhi_ref, o_ref):
    a = q_ref[0, :NK, :] + q_ref[2, :NK, :]
    b = q_ref[1, :NK, :] + q_ref[3, :NK, :]
    o_ref[0, :, :] = _bn_packed(a, glo_ref[...], blo_ref[...])
    o_ref[1, :, :] = _bn_packed(b, ghi_ref[...], bhi_ref[...])


KB = 6064


def _cls_body(a_ref, w_ref, gc_ref, bec_ref, wc2_ref, bc2_ref, o_ref, acc_ref):
    k = pl.program_id(0)

    @pl.when(k == 0)
    def _():
        acc_ref[...] = jnp.zeros_like(acc_ref)

    acc_ref[...] += lax.dot_general(
        a_ref[...], w_ref[...], (((0,), (0,)), ((), ())),
        preferred_element_type=jnp.float32)

    @pl.when(k == pl.num_programs(0) - 1)
    def _():
        z = acc_ref[...]
        m = jnp.mean(z, axis=0, keepdims=True)
        d = z - m
        v = jnp.mean(d * d, axis=0, keepdims=True)
        zn = d * lax.rsqrt(v + 1e-5) * gc_ref[...] + bec_ref[...]
        zn = jnp.maximum(zn, 0.0)
        o_ref[...] = (jnp.dot(zn, wc2_ref[...],
                              preferred_element_type=jnp.float32)
                      + bc2_ref[...])


def _bd4(a):
    return jnp.kron(jnp.eye(4, dtype=a.dtype), a)


def _tile4(v):
    return jnp.tile(v, 4).reshape(1, 128)


def _mm(x, w):
    xp = x.reshape(NK, 32)
    return pl.pallas_call(
        _mm_body,
        out_shape=jax.ShapeDtypeStruct((2, NPK, 128), jnp.float32),
    )(xp, _bd4(w[:, :FH]), _bd4(w[:, FH:]))


def _mid(p, g, be, w):
    q = p.reshape(4, NPK, 128)
    return pl.pallas_call(
        _mid_body,
        out_shape=jax.ShapeDtypeStruct((2, NPK, 128), jnp.float32),
    )(q, _bd4(w[:FH, :FH]), _bd4(w[FH:, :FH]),
      _bd4(w[:FH, FH:]), _bd4(w[FH:, FH:]),
      _tile4(g[:FH]), _tile4(g[FH:]), _tile4(be[:FH]), _tile4(be[FH:]))


def _last(p, g, be):
    q = p.reshape(4, NPK, 128)
    return pl.pallas_call(
        _last_body,
        out_shape=jax.ShapeDtypeStruct((2, NK, 128), jnp.float32),
    )(q, _tile4(g[:FH]), _tile4(g[FH:]), _tile4(be[:FH]), _tile4(be[FH:]))


def _classifier(at, wc1, gc, bec, wc2, bc2):
    nk = at.shape[0] // KB
    return pl.pallas_call(
        _cls_body,
        grid=(nk,),
        in_specs=[
            pl.BlockSpec((KB, B), lambda k: (k, 0)),
            pl.BlockSpec((KB, 128), lambda k: (k, 0)),
            pl.BlockSpec((1, 128), lambda k: (0, 0)),
            pl.BlockSpec((1, 128), lambda k: (0, 0)),
            pl.BlockSpec((128, 2), lambda k: (0, 0)),
            pl.BlockSpec((1, 2), lambda k: (0, 0)),
        ],
        out_specs=pl.BlockSpec((B, 2), lambda k: (0, 0)),
        out_shape=jax.ShapeDtypeStruct((B, 2), jnp.float32),
        scratch_shapes=[pltpu.VMEM((B, 128), jnp.float32)],
    )(at, wc1, gc.reshape(1, 128), bec.reshape(1, 128), wc2,
      bc2.reshape(1, 2))


def kernel(x, edge_index, edge_vals, W1, b1, g1, be1, W2, b2, g2, be2,
           W3, b3, g3, be3, Wc1, bc1, gc, bec, Wc2, bc2):
    ht = jnp.transpose(x, (0, 2, 1)).reshape(N, C_IN)
    ht8 = jnp.pad(ht, ((0, 0), (0, 8 - C_IN)))
    W1p = jnp.pad(W1, ((0, 8 - C_IN), (0, 0)))

    pad = E_PAD - E
    src = jnp.concatenate([edge_index[0], jnp.zeros((pad,), jnp.int32)])
    dst = jnp.concatenate([edge_index[1], jnp.zeros((pad,), jnp.int32)])
    vals = jnp.concatenate([edge_vals, jnp.zeros((pad,), jnp.float32)])
    src = src.reshape(NW, CPW, CH)
    dst = dst.reshape(NW, CPW, CH)
    vals = vals.reshape(NW, EPW)
    zeros = jnp.zeros((NP, FH), jnp.float32)

    h = _mm(ht8, W1p).reshape(2, NP, FH)
    p = _spmm_sc(h, src, dst, vals, zeros)
    h = _mid(p, g1, be1, W2).reshape(2, NP, FH)
    p = _spmm_sc(h, src, dst, vals, zeros)
    h = _mid(p, g2, be2, W3).reshape(2, NP, FH)
    p = _spmm_sc(h, src, dst, vals, zeros)
    hn = _last(p, g3, be3)

    h3 = jnp.concatenate(
        [hn[0].reshape(N, FH), hn[1].reshape(N, FH)], axis=1)
    at = h3.reshape(B, P * F).T
    return _classifier(at, Wc1, gc, bec, Wc2, bc2)


# speedup vs baseline: 1.9221x; 1.0399x over previous
import functools

import jax
import jax.numpy as jnp
from jax import lax
from jax.experimental import pallas as pl
from jax.experimental.pallas import tpu as pltpu
from jax.experimental.pallas import tpu_sc as plsc

B = 16
P = 1516
C_IN = 3
F = 64
N = B * P
E = N * 16

NC = 2
NS = 16
NW = NC * NS
CH = 128
CPW = 96
BL = 48
NB = CPW // BL
EPW = CPW * CH
E_PAD = NW * EPW
RPS = 1520
NP = NS * RPS

_mesh = plsc.VectorSubcoreMesh(core_axis_name="c", subcore_axis_name="s")

_GATHER_DNUMS = lax.GatherDimensionNumbers(
    offset_dims=(), collapsed_slice_dims=(0,), start_index_map=(0,))


def _bcast_lane(vv, l):
    idx = jnp.full((16, 1), l, jnp.int32)
    return lax.gather(vv, idx, _GATHER_DNUMS, (1,),
                      mode=lax.GatherScatterMode.PROMISE_IN_BOUNDS)


FH = F // 2


@functools.partial(
    pl.kernel,
    out_type=jax.ShapeDtypeStruct((NC, 2, NP, FH), jnp.float32),
    mesh=_mesh,
    scratch_types=[
        pltpu.VMEM((BL, CH), jnp.int32),
        pltpu.VMEM((BL, CH), jnp.int32),
        pltpu.VMEM((BL * CH,), jnp.float32),
        pltpu.VMEM((CH, FH), jnp.float32),
        pltpu.VMEM((CH, FH), jnp.float32),
        pltpu.VMEM((CH, FH), jnp.float32),
        pltpu.VMEM_SHARED((NP, FH), jnp.float32),
        pltpu.VMEM_SHARED((NP, FH), jnp.float32),
        pltpu.SemaphoreType.DMA,
        pltpu.SemaphoreType.DMA,
    ],
    compiler_params=pltpu.CompilerParams(use_tc_tiling_on_sc=False),
)
def _spmm_sc(h_hbm, src_hbm, dst_hbm, vals_hbm, zeros_hbm, out_hbm,
             src_v, dst_v, vals_v, rows0, rows1, rows2, hspm, acc,
             gsem, ssem):
    c = lax.axis_index("c")
    s = lax.axis_index("s")
    wid = s * NC + c

    def start_gather(jj, rows_ref):
        pltpu.async_copy(hspm.at[src_v.at[jj]], rows_ref, gsem)

    def wait_gather(jj, rows_ref):
        pltpu.make_async_copy(hspm.at[src_v.at[jj]], rows_ref, gsem).wait()

    def start_scatter(jj, rows_ref):
        pltpu.async_copy(rows_ref, acc.at[dst_v.at[jj]], ssem, add=True)

    def wait_scatter(jj, rows_ref):
        pltpu.make_async_copy(rows_ref, acc.at[dst_v.at[jj]],
                              ssem).wait()

    def scale(rows_ref, j):
        def scale_body(g, carry2):
            vv = vals_v[pl.ds(j * CH + g * 16, 16)]
            for l in range(16):
                v16 = _bcast_lane(vv, l)
                i = g * 16 + l
                for cb in range(FH // 16):
                    sl = (i, pl.ds(cb * 16, 16))
                    rows_ref[sl] = rows_ref[sl] * v16
            return carry2

        lax.fori_loop(0, CH // 16, scale_body, 0)

    def step(t, cur, prev):
        wait_gather(t, cur)
        scale(cur, t)
        start_scatter(t, cur)

        @pl.when(t >= 1)
        def _():
            wait_scatter(t - 1, prev)

        @pl.when(t + 2 < BL)
        def _():
            start_gather(t + 2, prev)

    def block_body(blk, carry):
        pltpu.sync_copy(src_hbm.at[wid, pl.ds(blk * BL, BL)], src_v)
        pltpu.sync_copy(dst_hbm.at[wid, pl.ds(blk * BL, BL)], dst_v)
        pltpu.sync_copy(vals_hbm.at[wid, pl.ds(blk * BL * CH, BL * CH)],
                        vals_v)
        start_gather(0, rows0)
        start_gather(1, rows1)

        def inner(m, carry1):
            t = 3 * m
            step(t, rows0, rows2)
            step(t + 1, rows1, rows0)
            step(t + 2, rows2, rows1)
            return carry1

        lax.fori_loop(0, BL // 3, inner, 0)
        wait_scatter(BL - 1, rows2)
        return carry

    for ph in range(2):
        pltpu.sync_copy(h_hbm.at[ph, pl.ds(s * RPS, RPS)],
                        hspm.at[pl.ds(s * RPS, RPS)])
        pltpu.sync_copy(zeros_hbm.at[pl.ds(s * RPS, RPS)],
                        acc.at[pl.ds(s * RPS, RPS)])
        plsc.subcore_barrier()

        lax.fori_loop(0, NB, block_body, 0)

        plsc.subcore_barrier()
        pltpu.sync_copy(acc.at[pl.ds(s * RPS, RPS)],
                        out_hbm.at[c, ph, pl.ds(s * RPS, RPS)])


NK = N * FH // 128
NPK = NP * FH // 128


def _bn_packed(a, gp, bep):
    m4 = jnp.mean(a, axis=0, keepdims=True)
    s4 = jnp.mean(a * a, axis=0, keepdims=True)
    mg = (m4[:, 0:32] + m4[:, 32:64] + m4[:, 64:96] + m4[:, 96:128]) * 0.25
    sg = (s4[:, 0:32] + s4[:, 32:64] + s4[:, 64:96] + s4[:, 96:128]) * 0.25
    var = sg - mg * mg
    scale = lax.rsqrt(var + 1e-5)
    mg = jnp.concatenate([mg, mg, mg, mg], axis=1)
    scale = jnp.concatenate([scale, scale, scale, scale], axis=1)
    return jnp.maximum((a - mg) * scale * gp + bep, 0.0)


def _mm_body(x_ref, wlo_ref, whi_ref, o_ref):
    xb = x_ref[...]
    o_ref[0, :NK, :] = jnp.dot(xb, wlo_ref[...],
                               preferred_element_type=jnp.float32)
    o_ref[1, :NK, :] = jnp.dot(xb, whi_ref[...],
                               preferred_element_type=jnp.float32)


def _mid_body(q_ref, wll_ref, whl_ref, wlh_ref, whh_ref,
              glo_ref, ghi_ref, blo_ref, bhi_ref, o_ref):
    a = q_ref[0, :NK, :] + q_ref[2, :NK, :]
    b = q_ref[1, :NK, :] + q_ref[3, :NK, :]
    an = _bn_packed(a, glo_ref[...], blo_ref[...])
    bn_ = _bn_packed(b, ghi_ref[...], bhi_ref[...])
    o_ref[0, :NK, :] = (
        jnp.dot(an, wll_ref[...], preferred_element_type=jnp.float32)
        + jnp.dot(bn_, whl_ref[...], preferred_element_type=jnp.float32))
    o_ref[1, :NK, :] = (
        jnp.dot(an, wlh_ref[...], preferred_element_type=jnp.float32)
        + jnp.dot(bn_, whh_ref[...], preferred_element_type=jnp.float32))


def _last_body(q_ref, glo_ref, ghi_ref, blo_ref, bhi_ref, o_ref):
    a = q_ref[0, :NK, :] + q_ref[2, :NK, :]
    b = q_ref[1, :NK, :] + q_ref[3, :NK, :]
    o_ref[0, :, :] = _bn_packed(a, glo_ref[...], blo_ref[...])
    o_ref[1, :, :] = _bn_packed(b, ghi_ref[...], bhi_ref[...])


KB = 6064


def _cls_body(a_ref, w_ref, gc_ref, bec_ref, wc2_ref, bc2_ref, o_ref, acc_ref):
    k = pl.program_id(0)

    @pl.when(k == 0)
    def _():
        acc_ref[...] = jnp.zeros_like(acc_ref)

    acc_ref[...] += lax.dot_general(
        a_ref[...], w_ref[...], (((0,), (0,)), ((), ())),
        preferred_element_type=jnp.float32)

    @pl.when(k == pl.num_programs(0) - 1)
    def _():
        z = acc_ref[...]
        m = jnp.mean(z, axis=0, keepdims=True)
        d = z - m
        v = jnp.mean(d * d, axis=0, keepdims=True)
        zn = d * lax.rsqrt(v + 1e-5) * gc_ref[...] + bec_ref[...]
        zn = jnp.maximum(zn, 0.0)
        o_ref[...] = (jnp.dot(zn, wc2_ref[...],
                              preferred_element_type=jnp.float32)
                      + bc2_ref[...])


def _bd4(a):
    return jnp.kron(jnp.eye(4, dtype=a.dtype), a)


def _tile4(v):
    return jnp.tile(v, 4).reshape(1, 128)


def _mm(x, w):
    xp = x.reshape(NK, 32)
    return pl.pallas_call(
        _mm_body,
        out_shape=jax.ShapeDtypeStruct((2, NPK, 128), jnp.float32),
    )(xp, _bd4(w[:, :FH]), _bd4(w[:, FH:]))


def _mid(p, g, be, w):
    q = p.reshape(4, NPK, 128)
    return pl.pallas_call(
        _mid_body,
        out_shape=jax.ShapeDtypeStruct((2, NPK, 128), jnp.float32),
    )(q, _bd4(w[:FH, :FH]), _bd4(w[FH:, :FH]),
      _bd4(w[:FH, FH:]), _bd4(w[FH:, FH:]),
      _tile4(g[:FH]), _tile4(g[FH:]), _tile4(be[:FH]), _tile4(be[FH:]))


def _last(p, g, be):
    q = p.reshape(4, NPK, 128)
    return pl.pallas_call(
        _last_body,
        out_shape=jax.ShapeDtypeStruct((2, NK, 128), jnp.float32),
    )(q, _tile4(g[:FH]), _tile4(g[FH:]), _tile4(be[:FH]), _tile4(be[FH:]))


def _classifier(at, wc1, gc, bec, wc2, bc2):
    nk = at.shape[0] // KB
    return pl.pallas_call(
        _cls_body,
        grid=(nk,),
        in_specs=[
            pl.BlockSpec((KB, B), lambda k: (k, 0)),
            pl.BlockSpec((KB, 128), lambda k: (k, 0)),
            pl.BlockSpec((1, 128), lambda k: (0, 0)),
            pl.BlockSpec((1, 128), lambda k: (0, 0)),
            pl.BlockSpec((128, 2), lambda k: (0, 0)),
            pl.BlockSpec((1, 2), lambda k: (0, 0)),
        ],
        out_specs=pl.BlockSpec((B, 2), lambda k: (0, 0)),
        out_shape=jax.ShapeDtypeStruct((B, 2), jnp.float32),
        scratch_shapes=[pltpu.VMEM((B, 128), jnp.float32)],
    )(at, wc1, gc.reshape(1, 128), bec.reshape(1, 128), wc2,
      bc2.reshape(1, 2))


def kernel(x, edge_index, edge_vals, W1, b1, g1, be1, W2, b2, g2, be2,
           W3, b3, g3, be3, Wc1, bc1, gc, bec, Wc2, bc2):
    ht = jnp.transpose(x, (0, 2, 1)).reshape(N, C_IN)
    ht8 = jnp.pad(ht, ((0, 0), (0, 8 - C_IN)))
    W1p = jnp.pad(W1, ((0, 8 - C_IN), (0, 0)))

    pad = E_PAD - E
    src = jnp.concatenate([edge_index[0], jnp.zeros((pad,), jnp.int32)])
    dst = jnp.concatenate([edge_index[1], jnp.zeros((pad,), jnp.int32)])
    vals = jnp.concatenate([edge_vals, jnp.zeros((pad,), jnp.float32)])
    src = src.reshape(NW, CPW, CH)
    dst = dst.reshape(NW, CPW, CH)
    vals = vals.reshape(NW, EPW)
    zeros = jnp.zeros((NP, FH), jnp.float32)

    h = _mm(ht8, W1p).reshape(2, NP, FH)
    p = _spmm_sc(h, src, dst, vals, zeros)
    h = _mid(p, g1, be1, W2).reshape(2, NP, FH)
    p = _spmm_sc(h, src, dst, vals, zeros)
    h = _mid(p, g2, be2, W3).reshape(2, NP, FH)
    p = _spmm_sc(h, src, dst, vals, zeros)
    hn = _last(p, g3, be3)

    h3 = jnp.concatenate(
        [hn[0].reshape(N, FH), hn[1].reshape(N, FH)], axis=1)
    at = h3.reshape(B, P * F).T
    return _classifier(at, Wc1, gc, bec, Wc2, bc2)
